# baseline (speedup 1.0000x reference)
import functools

import jax
import jax.numpy as jnp
from jax.experimental import pallas as pl
from jax.experimental.pallas import tpu as pltpu

N = 2048
D_IN = 512
B_HID = 4096
CODE = 64
NUM_GRIDS = 8
GRID_MIN, GRID_MAX = -2.0, 2.0
D_OUT = 2 * D_IN

BM = 256
NB = N // BM

_PREC = jax.lax.Precision.HIGHEST


def _dot(a, b):
    return jax.lax.dot_general(
        a, b, (((1,), (0,)), ((), ())),
        precision=_PREC, preferred_element_type=jnp.float32)


def _fused_kernel(x_ref, g_ref, w1_ref, b1_ref, w2_ref, b2_ref,
                  lnw_ref, lnb_ref, w3_ref, b3_ref,
                  code_ref, out_ref, t2_scr):
    p = pl.program_id(0)
    i = pl.program_id(1)

    @pl.when(p == 0)
    def _phase0():
        y = _dot(g_ref[...], x_ref[...])
        h = jnp.maximum(_dot(y, w1_ref[...]) + b1_ref[...], 0.0)
        t2_scr[pl.ds(i * BM, BM), :] = _dot(h, w2_ref[...])
        code_ref[...] = jnp.zeros_like(code_ref)
        out_ref[...] = jnp.zeros_like(out_ref)

    @pl.when(p == 1)
    def _phase1():
        feat = _dot(g_ref[...], t2_scr[...]) + b2_ref[...]
        code = jnp.tanh(10.0 * feat)
        code_ref[...] = code

        mu = jnp.mean(code, axis=-1, keepdims=True)
        var = jnp.mean((code - mu) ** 2, axis=-1, keepdims=True)
        y = (code - mu) * jax.lax.rsqrt(var + 1e-5) * lnw_ref[...] + lnb_ref[...]

        denom = (GRID_MAX - GRID_MIN) / (NUM_GRIDS - 1)
        inv_denom = 1.0 / denom
        acc = jnp.zeros((BM, D_OUT), dtype=jnp.float32)
        for g in range(NUM_GRIDS):
            grid_g = GRID_MIN + g * denom
            t = (y - grid_g) * inv_denom
            acc = acc + _dot(jnp.exp(-(t * t)), w3_ref[g])
        out_ref[...] = jnp.maximum(acc + b3_ref[...], 0.0)


@jax.jit
def kernel(x, G, W1, b1, W2, b2, ln_w, ln_b, W3, b3):
    W3g = W3.reshape(CODE, NUM_GRIDS, D_OUT).transpose(1, 0, 2)
    row = lambda v: v.reshape(1, -1)

    full = lambda shape: pl.BlockSpec(shape, lambda p, i: (0,) * len(shape))
    gspec = pl.BlockSpec((BM, N), lambda p, i: (i, 0))

    code, feat_out = pl.pallas_call(
        _fused_kernel,
        grid=(2, NB),
        in_specs=[
            full((N, D_IN)),
            gspec,
            full((D_IN, B_HID)),
            full((1, B_HID)),
            full((B_HID, CODE)),
            full((1, CODE)),
            full((1, CODE)),
            full((1, CODE)),
            full((NUM_GRIDS, CODE, D_OUT)),
            full((1, D_OUT)),
        ],
        out_specs=[
            pl.BlockSpec((BM, CODE), lambda p, i: (i, 0)),
            pl.BlockSpec((BM, D_OUT), lambda p, i: (i, 0)),
        ],
        out_shape=[
            jax.ShapeDtypeStruct((N, CODE), jnp.float32),
            jax.ShapeDtypeStruct((N, D_OUT), jnp.float32),
        ],
        scratch_shapes=[pltpu.VMEM((N, CODE), jnp.float32)],
        compiler_params=pltpu.CompilerParams(
            dimension_semantics=("arbitrary", "arbitrary")),
    )(x, G, W1, row(b1), W2, row(b2), row(ln_w), row(ln_b), W3g, row(b3))
    return (code, feat_out)


# speedup vs baseline: 4.1696x; 4.1696x over previous
import functools

import jax
import jax.numpy as jnp
from jax.experimental import pallas as pl
from jax.experimental.pallas import tpu as pltpu

N = 2048
D_IN = 512
B_HID = 4096
CODE = 64
NUM_GRIDS = 8
GRID_MIN, GRID_MAX = -2.0, 2.0
D_OUT = 2 * D_IN

BM = 256
NB = N // BM

_PREC = jax.lax.Precision.DEFAULT


def _dot(a, b):
    return jax.lax.dot_general(
        a, b, (((1,), (0,)), ((), ())),
        precision=_PREC, preferred_element_type=jnp.float32)


def _fused_kernel(x_ref, g_ref, w1_ref, b1_ref, w2_ref, b2_ref,
                  lnw_ref, lnb_ref, w3_ref, b3_ref,
                  code_ref, out_ref, t2_scr):
    p = pl.program_id(0)
    i = pl.program_id(1)

    @pl.when(p == 0)
    def _phase0():
        y = _dot(g_ref[...], x_ref[...])
        h = jnp.maximum(_dot(y, w1_ref[...]) + b1_ref[...], 0.0)
        t2_scr[pl.ds(i * BM, BM), :] = _dot(h, w2_ref[...])
        code_ref[...] = jnp.zeros_like(code_ref)
        out_ref[...] = jnp.zeros_like(out_ref)

    @pl.when(p == 1)
    def _phase1():
        feat = _dot(g_ref[...], t2_scr[...]) + b2_ref[...]
        code = jnp.tanh(10.0 * feat)
        code_ref[...] = code

        mu = jnp.mean(code, axis=-1, keepdims=True)
        var = jnp.mean((code - mu) ** 2, axis=-1, keepdims=True)
        y = (code - mu) * jax.lax.rsqrt(var + 1e-5) * lnw_ref[...] + lnb_ref[...]

        denom = (GRID_MAX - GRID_MIN) / (NUM_GRIDS - 1)
        inv_denom = 1.0 / denom
        acc = jnp.zeros((BM, D_OUT), dtype=jnp.float32)
        for g in range(NUM_GRIDS):
            grid_g = GRID_MIN + g * denom
            t = (y - grid_g) * inv_denom
            acc = acc + _dot(jnp.exp(-(t * t)), w3_ref[g])
        out_ref[...] = jnp.maximum(acc + b3_ref[...], 0.0)


@jax.jit
def kernel(x, G, W1, b1, W2, b2, ln_w, ln_b, W3, b3):
    W3g = W3.reshape(CODE, NUM_GRIDS, D_OUT).transpose(1, 0, 2)
    row = lambda v: v.reshape(1, -1)

    full = lambda shape: pl.BlockSpec(shape, lambda p, i: (0,) * len(shape))
    gspec = pl.BlockSpec((BM, N), lambda p, i: (i, 0))

    code, feat_out = pl.pallas_call(
        _fused_kernel,
        grid=(2, NB),
        in_specs=[
            full((N, D_IN)),
            gspec,
            full((D_IN, B_HID)),
            full((1, B_HID)),
            full((B_HID, CODE)),
            full((1, CODE)),
            full((1, CODE)),
            full((1, CODE)),
            full((NUM_GRIDS, CODE, D_OUT)),
            full((1, D_OUT)),
        ],
        out_specs=[
            pl.BlockSpec((BM, CODE), lambda p, i: (i, 0)),
            pl.BlockSpec((BM, D_OUT), lambda p, i: (i, 0)),
        ],
        out_shape=[
            jax.ShapeDtypeStruct((N, CODE), jnp.float32),
            jax.ShapeDtypeStruct((N, D_OUT), jnp.float32),
        ],
        scratch_shapes=[pltpu.VMEM((N, CODE), jnp.float32)],
        compiler_params=pltpu.CompilerParams(
            dimension_semantics=("arbitrary", "arbitrary")),
    )(x, G, W1, row(b1), W2, row(b2), row(ln_w), row(ln_b), W3g, row(b3))
    return (code, feat_out)


# speedup vs baseline: 4.3389x; 1.0406x over previous
import functools

import jax
import jax.numpy as jnp
from jax.experimental import pallas as pl
from jax.experimental.pallas import tpu as pltpu

N = 2048
D_IN = 512
B_HID = 4096
CODE = 64
NUM_GRIDS = 8
GRID_MIN, GRID_MAX = -2.0, 2.0
D_OUT = 2 * D_IN

BM = 512
NB = N // BM

_PREC = jax.lax.Precision.DEFAULT


def _dot(a, b):
    return jax.lax.dot_general(
        a, b, (((1,), (0,)), ((), ())),
        precision=_PREC, preferred_element_type=jnp.float32)


def _fused_kernel(x_ref, g_ref, w1_ref, b1_ref, w2_ref, b2_ref,
                  lnw_ref, lnb_ref, w3_ref, b3_ref,
                  code_ref, out_ref, t2_scr):
    p = pl.program_id(0)
    i = pl.program_id(1)

    @pl.when(p == 0)
    def _phase0():
        y = _dot(g_ref[...], x_ref[...])
        h = jnp.maximum(_dot(y, w1_ref[...]) + b1_ref[...], 0.0)
        t2_scr[pl.ds(i * BM, BM), :] = _dot(h, w2_ref[...])

    @pl.when(p == 1)
    def _phase1():
        feat = _dot(g_ref[...], t2_scr[...]) + b2_ref[...]
        code = jnp.tanh(10.0 * feat)
        code_ref[...] = code

        mu = jnp.mean(code, axis=-1, keepdims=True)
        var = jnp.mean((code - mu) ** 2, axis=-1, keepdims=True)
        y = (code - mu) * jax.lax.rsqrt(var + 1e-5) * lnw_ref[...] + lnb_ref[...]

        denom = (GRID_MAX - GRID_MIN) / (NUM_GRIDS - 1)
        inv_denom = 1.0 / denom
        acc = jnp.zeros((BM, D_OUT), dtype=jnp.float32)
        for g in range(NUM_GRIDS):
            grid_g = GRID_MIN + g * denom
            t = (y - grid_g) * inv_denom
            acc = acc + _dot(jnp.exp(-(t * t)), w3_ref[g])
        out_ref[...] = jnp.maximum(acc + b3_ref[...], 0.0)


@jax.jit
def kernel(x, G, W1, b1, W2, b2, ln_w, ln_b, W3, b3):
    W3g = W3.reshape(CODE, NUM_GRIDS, D_OUT).transpose(1, 0, 2)
    row = lambda v: v.reshape(1, -1)

    full = lambda shape: pl.BlockSpec(shape, lambda p, i: (0,) * len(shape))
    gspec = pl.BlockSpec((BM, N), lambda p, i: (i, 0))

    code, feat_out = pl.pallas_call(
        _fused_kernel,
        grid=(2, NB),
        in_specs=[
            full((N, D_IN)),
            gspec,
            full((D_IN, B_HID)),
            full((1, B_HID)),
            full((B_HID, CODE)),
            full((1, CODE)),
            full((1, CODE)),
            full((1, CODE)),
            full((NUM_GRIDS, CODE, D_OUT)),
            full((1, D_OUT)),
        ],
        out_specs=[
            pl.BlockSpec((BM, CODE), lambda p, i: (i, 0)),
            pl.BlockSpec((BM, D_OUT), lambda p, i: (i, 0)),
        ],
        out_shape=[
            jax.ShapeDtypeStruct((N, CODE), jnp.float32),
            jax.ShapeDtypeStruct((N, D_OUT), jnp.float32),
        ],
        scratch_shapes=[pltpu.VMEM((N, CODE), jnp.float32)],
        compiler_params=pltpu.CompilerParams(
            dimension_semantics=("arbitrary", "arbitrary")),
    )(x, G, W1, row(b1), W2, row(b2), row(ln_w), row(ln_b), W3g, row(b3))
    return (code, feat_out)


# speedup vs baseline: 4.4729x; 1.0309x over previous
import jax
import jax.numpy as jnp
from jax.experimental import pallas as pl
from jax.experimental.pallas import tpu as pltpu

N = 2048
D_IN = 512
B_HID = 4096
CODE = 64
NUM_GRIDS = 8
GRID_MIN, GRID_MAX = -2.0, 2.0
D_OUT = 2 * D_IN

BM = 512
NB = N // BM


def _dot(a, b):
    return jax.lax.dot_general(
        a, b, (((1,), (0,)), ((), ())),
        preferred_element_type=jnp.float32)


def _fused_kernel(x_ref, g_ref, gc_ref, w1_ref, b1_ref, w2_ref, b2_ref,
                  lnw_ref, lnb_ref, w3_ref, b3_ref,
                  code_ref, out_ref, feat_scr):
    p = pl.program_id(0)
    i = pl.program_id(1)

    @pl.when(p == 0)
    def _phase0():
        y = _dot(g_ref[...], x_ref[...])
        h = jnp.maximum(_dot(y, w1_ref[...]) + b1_ref[...], 0.0)
        t2 = _dot(h, w2_ref[...])
        contrib = _dot(gc_ref[...], t2)

        @pl.when(i == 0)
        def _init():
            feat_scr[...] = contrib

        @pl.when(i > 0)
        def _accum():
            feat_scr[...] += contrib

    @pl.when(p == 1)
    def _phase1():
        feat = feat_scr[pl.ds(i * BM, BM), :] + b2_ref[...]
        code = jnp.tanh(10.0 * feat)
        code_ref[...] = code

        mu = jnp.mean(code, axis=-1, keepdims=True)
        var = jnp.mean((code - mu) ** 2, axis=-1, keepdims=True)
        y = (code - mu) * jax.lax.rsqrt(var + 1e-5) * lnw_ref[...] + lnb_ref[...]

        denom = (GRID_MAX - GRID_MIN) / (NUM_GRIDS - 1)
        inv_denom = 1.0 / denom
        acc = jnp.zeros((BM, D_OUT), dtype=jnp.float32)
        for g in range(NUM_GRIDS):
            grid_g = GRID_MIN + g * denom
            t = (y - grid_g) * inv_denom
            acc = acc + _dot(jnp.exp(-(t * t)), w3_ref[g])
        out_ref[...] = jnp.maximum(acc + b3_ref[...], 0.0)


@jax.jit
def kernel(x, G, W1, b1, W2, b2, ln_w, ln_b, W3, b3):
    W3g = W3.reshape(CODE, NUM_GRIDS, D_OUT).transpose(1, 0, 2)
    row = lambda v: v.reshape(1, -1)

    full = lambda shape: pl.BlockSpec(shape, lambda p, i: (0,) * len(shape))
    grow = pl.BlockSpec((BM, N), lambda p, i: (jnp.where(p == 0, i, NB - 1), 0))
    gcol = pl.BlockSpec((N, BM), lambda p, i: (0, jnp.where(p == 0, i, NB - 1)))

    code, feat_out = pl.pallas_call(
        _fused_kernel,
        grid=(2, NB),
        in_specs=[
            full((N, D_IN)),
            grow,
            gcol,
            full((D_IN, B_HID)),
            full((1, B_HID)),
            full((B_HID, CODE)),
            full((1, CODE)),
            full((1, CODE)),
            full((1, CODE)),
            full((NUM_GRIDS, CODE, D_OUT)),
            full((1, D_OUT)),
        ],
        out_specs=[
            pl.BlockSpec((BM, CODE), lambda p, i: (i, 0)),
            pl.BlockSpec((BM, D_OUT), lambda p, i: (i, 0)),
        ],
        out_shape=[
            jax.ShapeDtypeStruct((N, CODE), jnp.float32),
            jax.ShapeDtypeStruct((N, D_OUT), jnp.float32),
        ],
        scratch_shapes=[pltpu.VMEM((N, CODE), jnp.float32)],
        compiler_params=pltpu.CompilerParams(
            dimension_semantics=("arbitrary", "arbitrary")),
    )(x, G, G, W1, row(b1), W2, row(b2), row(ln_w), row(ln_b), W3g, row(b3))
    return (code, feat_out)


# speedup vs baseline: 4.6992x; 1.0506x over previous
import jax
import jax.numpy as jnp
import numpy as np
from jax.experimental import pallas as pl
from jax.experimental.pallas import tpu as pltpu

N = 2048
D_IN = 512
B_HID = 4096
CODE = 64
NUM_GRIDS = 8
GRID_MIN, GRID_MAX = -2.0, 2.0
D_OUT = 2 * D_IN
KAN_K = CODE * NUM_GRIDS

BM = 512
NB = N // BM

_DENOM = (GRID_MAX - GRID_MIN) / (NUM_GRIDS - 1)


def _dot(a, b):
    return jax.lax.dot_general(
        a, b, (((1,), (0,)), ((), ())),
        preferred_element_type=jnp.float32)


def _bf16(v):
    return v.astype(jnp.bfloat16)


def _fused_kernel(x_ref, g_ref, gc_ref, w1_ref, b1_ref, w2_ref, b2_ref,
                  lnw_ref, lnb_ref, w3_ref, b3_ref,
                  code_ref, out_ref,
                  feat_scr, xb_scr, w1b_scr, w2b_scr, w3b_scr):
    p = pl.program_id(0)
    i = pl.program_id(1)

    @pl.when((p == 0) & (i == 0))
    def _pack_weights():
        xb_scr[...] = _bf16(x_ref[...])
        w1b_scr[...] = _bf16(w1_ref[...])
        w2b_scr[...] = _bf16(w2_ref[...])
        w3b_scr[...] = _bf16(w3_ref[...])

    @pl.when(p == 0)
    def _phase0():
        y = _dot(_bf16(g_ref[...]), xb_scr[...])
        h = jnp.maximum(_dot(_bf16(y), w1b_scr[...]) + b1_ref[...], 0.0)
        t2 = _dot(_bf16(h), w2b_scr[...])
        contrib = _dot(_bf16(gc_ref[...]), _bf16(t2))

        @pl.when(i == 0)
        def _init():
            feat_scr[...] = contrib

        @pl.when(i > 0)
        def _accum():
            feat_scr[...] += contrib

    @pl.when(p == 1)
    def _phase1():
        feat = feat_scr[pl.ds(i * BM, BM), :] + b2_ref[...]
        code = jnp.tanh(10.0 * feat)
        code_ref[...] = code

        mu = jnp.mean(code, axis=-1, keepdims=True)
        var = jnp.mean((code - mu) ** 2, axis=-1, keepdims=True)
        y = (code - mu) * jax.lax.rsqrt(var + 1e-5) * lnw_ref[...] + lnb_ref[...]

        yt = jnp.tile(y, (1, NUM_GRIDS))
        gidx = jax.lax.broadcasted_iota(jnp.int32, (1, KAN_K), 1) // CODE
        gcols = GRID_MIN + gidx.astype(jnp.float32) * _DENOM
        t = (yt - gcols) * (1.0 / _DENOM)
        rbf = jnp.exp(-(t * t))
        acc = _dot(_bf16(rbf), w3b_scr[...])
        out_ref[...] = jnp.maximum(acc + b3_ref[...], 0.0)


@jax.jit
def kernel(x, G, W1, b1, W2, b2, ln_w, ln_b, W3, b3):
    W3p = W3.reshape(CODE, NUM_GRIDS, D_OUT).transpose(1, 0, 2).reshape(KAN_K, D_OUT)
    row = lambda v: v.reshape(1, -1)

    full = lambda shape: pl.BlockSpec(shape, lambda p, i: (0,) * len(shape))
    grow = pl.BlockSpec((BM, N), lambda p, i: (jnp.where(p == 0, i, NB - 1), 0))
    gcol = pl.BlockSpec((N, BM), lambda p, i: (0, jnp.where(p == 0, i, NB - 1)))

    code, feat_out = pl.pallas_call(
        _fused_kernel,
        grid=(2, NB),
        in_specs=[
            full((N, D_IN)),
            grow,
            gcol,
            full((D_IN, B_HID)),
            full((1, B_HID)),
            full((B_HID, CODE)),
            full((1, CODE)),
            full((1, CODE)),
            full((1, CODE)),
            full((KAN_K, D_OUT)),
            full((1, D_OUT)),
        ],
        out_specs=[
            pl.BlockSpec((BM, CODE), lambda p, i: (i, 0)),
            pl.BlockSpec((BM, D_OUT), lambda p, i: (i, 0)),
        ],
        out_shape=[
            jax.ShapeDtypeStruct((N, CODE), jnp.float32),
            jax.ShapeDtypeStruct((N, D_OUT), jnp.float32),
        ],
        scratch_shapes=[
            pltpu.VMEM((N, CODE), jnp.float32),
            pltpu.VMEM((N, D_IN), jnp.bfloat16),
            pltpu.VMEM((D_IN, B_HID), jnp.bfloat16),
            pltpu.VMEM((B_HID, CODE), jnp.bfloat16),
            pltpu.VMEM((KAN_K, D_OUT), jnp.bfloat16),
        ],
        compiler_params=pltpu.CompilerParams(
            dimension_semantics=("arbitrary", "arbitrary")),
    )(x, G, G, W1, row(b1), W2, row(b2), row(ln_w), row(ln_b), W3p, row(b3))
    return (code, feat_out)


# speedup vs baseline: 5.1061x; 1.0866x over previous
import jax
import jax.numpy as jnp
from jax.experimental import pallas as pl
from jax.experimental.pallas import tpu as pltpu

N = 2048
D_IN = 512
B_HID = 4096
CODE = 64
NUM_GRIDS = 8
GRID_MIN, GRID_MAX = -2.0, 2.0
D_OUT = 2 * D_IN
KAN_K = CODE * NUM_GRIDS

BM = 512
NA = N // BM
BH_CHUNK = 1024
NJ = B_HID // BH_CHUNK
ND = N // BM
T_A, T_B = NA, NA + NJ
T_TOTAL = NA + NJ + ND

_DENOM = (GRID_MAX - GRID_MIN) / (NUM_GRIDS - 1)


def _dot(a, b):
    return jax.lax.dot_general(
        a, b, (((1,), (0,)), ((), ())),
        preferred_element_type=jnp.float32)


def _bf16(v):
    return v.astype(jnp.bfloat16)


def _fused_kernel(gc_ref, x_ref, w1_ref, b1_ref, w2_ref, b2_ref,
                  lnw_ref, lnb_ref, w3_ref, b3_ref,
                  code_ref, out_ref,
                  g_scr, y_scr, w1b_scr, w2b_scr, w3b_scr, t2_scr, t2b_scr):
    t = pl.program_id(0)

    @pl.when(t < T_A)
    def _phase_a():
        c = t
        gcb = _bf16(gc_ref[...])
        g_scr[:, pl.ds(c * BM, BM)] = gcb
        w1b_scr[:, pl.ds(c * BH_CHUNK, BH_CHUNK)] = _bf16(w1_ref[...])
        w2b_scr[pl.ds(c * BH_CHUNK, BH_CHUNK), :] = _bf16(w2_ref[...])
        contrib = _dot(gcb, _bf16(x_ref[...]))

        @pl.when(c == 0)
        def _init():
            y_scr[...] = contrib

        @pl.when(c > 0)
        def _accum():
            y_scr[...] += contrib

    @pl.when((t >= T_A) & (t < T_B))
    def _phase_b():
        j = t - T_A

        @pl.when(j == 0)
        def _pack_w3():
            w3b_scr[...] = _bf16(w3_ref[...])

        w1j = w1b_scr[:, pl.ds(j * BH_CHUNK, BH_CHUNK)]
        b1j = b1_ref[:, pl.ds(j * BH_CHUNK, BH_CHUNK)]
        h = jnp.maximum(_dot(_bf16(y_scr[...]), w1j) + b1j, 0.0)
        t2c = _dot(_bf16(h), w2b_scr[pl.ds(j * BH_CHUNK, BH_CHUNK), :])

        @pl.when(j == 0)
        def _init():
            t2_scr[...] = t2c

        @pl.when(j > 0)
        def _accum():
            t2_scr[...] += t2c

    @pl.when(t >= T_B)
    def _phase_d():
        i = t - T_B

        @pl.when(i == 0)
        def _pack_t2():
            t2b_scr[...] = _bf16(t2_scr[...])

        feat = _dot(g_scr[pl.ds(i * BM, BM), :], t2b_scr[...]) + b2_ref[...]
        code = jnp.tanh(10.0 * feat)
        code_ref[...] = code

        mu = jnp.mean(code, axis=-1, keepdims=True)
        var = jnp.mean((code - mu) ** 2, axis=-1, keepdims=True)
        y = (code - mu) * jax.lax.rsqrt(var + 1e-5) * lnw_ref[...] + lnb_ref[...]

        yt = jnp.tile(y, (1, NUM_GRIDS))
        gidx = jax.lax.broadcasted_iota(jnp.int32, (1, KAN_K), 1) // CODE
        gcols = GRID_MIN + gidx.astype(jnp.float32) * _DENOM
        tt = (yt - gcols) * (1.0 / _DENOM)
        rbf = jnp.exp(-(tt * tt))
        acc = _dot(_bf16(rbf), w3b_scr[...])
        out_ref[...] = jnp.maximum(acc + b3_ref[...], 0.0)


@jax.jit
def kernel(x, G, W1, b1, W2, b2, ln_w, ln_b, W3, b3):
    W3p = W3.reshape(CODE, NUM_GRIDS, D_OUT).transpose(1, 0, 2).reshape(KAN_K, D_OUT)
    row = lambda v: v.reshape(1, -1)

    full = lambda shape: pl.BlockSpec(shape, lambda t: (0,) * len(shape))

    code, feat_out = pl.pallas_call(
        _fused_kernel,
        grid=(T_TOTAL,),
        in_specs=[
            pl.BlockSpec((N, BM), lambda t: (0, jnp.minimum(t, NA - 1))),
            pl.BlockSpec((BM, D_IN), lambda t: (jnp.minimum(t, NA - 1), 0)),
            pl.BlockSpec((D_IN, BH_CHUNK), lambda t: (0, jnp.minimum(t, NJ - 1))),
            full((1, B_HID)),
            pl.BlockSpec((BH_CHUNK, CODE), lambda t: (jnp.minimum(t, NJ - 1), 0)),
            full((1, CODE)),
            full((1, CODE)),
            full((1, CODE)),
            full((KAN_K, D_OUT)),
            full((1, D_OUT)),
        ],
        out_specs=[
            pl.BlockSpec((BM, CODE), lambda t: (jnp.maximum(t - T_B, 0), 0)),
            pl.BlockSpec((BM, D_OUT), lambda t: (jnp.maximum(t - T_B, 0), 0)),
        ],
        out_shape=[
            jax.ShapeDtypeStruct((N, CODE), jnp.float32),
            jax.ShapeDtypeStruct((N, D_OUT), jnp.float32),
        ],
        scratch_shapes=[
            pltpu.VMEM((N, N), jnp.bfloat16),
            pltpu.VMEM((N, D_IN), jnp.float32),
            pltpu.VMEM((D_IN, B_HID), jnp.bfloat16),
            pltpu.VMEM((B_HID, CODE), jnp.bfloat16),
            pltpu.VMEM((KAN_K, D_OUT), jnp.bfloat16),
            pltpu.VMEM((N, CODE), jnp.float32),
            pltpu.VMEM((N, CODE), jnp.bfloat16),
        ],
        compiler_params=pltpu.CompilerParams(
            dimension_semantics=("arbitrary",)),
    )(G, x, W1, row(b1), W2, row(b2), row(ln_w), row(ln_b), W3p, row(b3))
    return (code, feat_out)


# speedup vs baseline: 5.3934x; 1.0563x over previous
import jax
import jax.numpy as jnp
from jax.experimental import pallas as pl
from jax.experimental.pallas import tpu as pltpu

N = 2048
N2 = N // 2
D_IN = 512
B_HID = 4096
CODE = 64
NUM_GRIDS = 8
GRID_MIN, GRID_MAX = -2.0, 2.0
D_OUT = 2 * D_IN
KAN_K = CODE * NUM_GRIDS

BM = 512
NA = N // BM
BH_CHUNK = 1024
NJ = B_HID // BH_CHUNK
W3_CHUNK = D_OUT // NJ
ND = N // BM
T_A, T_B = NA, NA + NJ
T_TOTAL = NA + NJ + ND

_DENOM = (GRID_MAX - GRID_MIN) / (NUM_GRIDS - 1)


def _dot(a, b):
    return jax.lax.dot_general(
        a, b, (((1,), (0,)), ((), ())),
        preferred_element_type=jnp.float32)


def _bf16(v):
    return v.astype(jnp.bfloat16)


def _fused_kernel(gt_ref, gb_ref, x_ref, w1_ref, b1_ref, w2_ref, b2_ref,
                  lnw_ref, lnb_ref, w3_ref, b3_ref,
                  code_ref, out_ref,
                  g_scr, y_scr, yb_scr, w3b_scr, t2_scr, t2b_scr):
    t = pl.program_id(0)

    @pl.when(t < T_A)
    def _phase_a():
        c = t
        gtb = _bf16(gt_ref[...])
        gbb = _bf16(gb_ref[...])
        g_scr[0:N2, pl.ds(c * BM, BM)] = gtb
        g_scr[N2:N, pl.ds(c * BM, BM)] = gbb
        xcb = _bf16(x_ref[...])
        top = _dot(gtb, xcb)
        bot = _dot(gbb, xcb)

        @pl.when(c == 0)
        def _init():
            y_scr[0:N2, :] = top
            y_scr[N2:N, :] = bot

        @pl.when(c > 0)
        def _accum():
            y_scr[0:N2, :] += top
            y_scr[N2:N, :] += bot

    @pl.when((t >= T_A) & (t < T_B))
    def _phase_b():
        j = t - T_A

        @pl.when(j == 0)
        def _pack_y():
            yb_scr[...] = _bf16(y_scr[...])

        w3b_scr[:, pl.ds(j * W3_CHUNK, W3_CHUNK)] = _bf16(w3_ref[...])

        b1j = b1_ref[:, pl.ds(j * BH_CHUNK, BH_CHUNK)]
        h = jnp.maximum(_dot(yb_scr[...], _bf16(w1_ref[...])) + b1j, 0.0)
        t2c = _dot(_bf16(h), _bf16(w2_ref[...]))

        @pl.when(j == 0)
        def _init():
            t2_scr[...] = t2c

        @pl.when(j > 0)
        def _accum():
            t2_scr[...] += t2c

    @pl.when(t >= T_B)
    def _phase_d():
        i = t - T_B

        @pl.when(i == 0)
        def _pack_t2():
            t2b_scr[...] = _bf16(t2_scr[...])

        feat = _dot(g_scr[pl.ds(i * BM, BM), :], t2b_scr[...]) + b2_ref[...]
        code = jnp.tanh(10.0 * feat)
        code_ref[...] = code

        mu = jnp.mean(code, axis=-1, keepdims=True)
        var = jnp.mean((code - mu) ** 2, axis=-1, keepdims=True)
        y = (code - mu) * jax.lax.rsqrt(var + 1e-5) * lnw_ref[...] + lnb_ref[...]

        yt = jnp.tile(y, (1, NUM_GRIDS))
        gidx = jax.lax.broadcasted_iota(jnp.int32, (1, KAN_K), 1) // CODE
        gcols = GRID_MIN + gidx.astype(jnp.float32) * _DENOM
        tt = (yt - gcols) * (1.0 / _DENOM)
        rbf = jnp.exp(-(tt * tt))
        acc = _dot(_bf16(rbf), w3b_scr[...])
        out_ref[...] = jnp.maximum(acc + b3_ref[...], 0.0)


@jax.jit
def kernel(x, G, W1, b1, W2, b2, ln_w, ln_b, W3, b3):
    W3p = W3.reshape(CODE, NUM_GRIDS, D_OUT).transpose(1, 0, 2).reshape(KAN_K, D_OUT)
    row = lambda v: v.reshape(1, -1)

    full = lambda shape: pl.BlockSpec(shape, lambda t: (0,) * len(shape))
    a_idx = lambda t: jnp.minimum(t, NA - 1)
    b_idx = lambda t: jnp.clip(t - T_A, 0, NJ - 1)

    code, feat_out = pl.pallas_call(
        _fused_kernel,
        grid=(T_TOTAL,),
        in_specs=[
            pl.BlockSpec((N2, BM), lambda t: (0, jnp.minimum(t, NA - 1))),
            pl.BlockSpec((N2, BM), lambda t: (1, jnp.minimum(t, NA - 1))),
            pl.BlockSpec((BM, D_IN), lambda t: (jnp.minimum(t, NA - 1), 0)),
            pl.BlockSpec((D_IN, BH_CHUNK), lambda t: (0, jnp.clip(t - T_A, 0, NJ - 1))),
            full((1, B_HID)),
            pl.BlockSpec((BH_CHUNK, CODE), lambda t: (jnp.clip(t - T_A, 0, NJ - 1), 0)),
            full((1, CODE)),
            full((1, CODE)),
            full((1, CODE)),
            pl.BlockSpec((KAN_K, W3_CHUNK), lambda t: (0, jnp.clip(t - T_A, 0, NJ - 1))),
            full((1, D_OUT)),
        ],
        out_specs=[
            pl.BlockSpec((BM, CODE), lambda t: (jnp.maximum(t - T_B, 0), 0)),
            pl.BlockSpec((BM, D_OUT), lambda t: (jnp.maximum(t - T_B, 0), 0)),
        ],
        out_shape=[
            jax.ShapeDtypeStruct((N, CODE), jnp.float32),
            jax.ShapeDtypeStruct((N, D_OUT), jnp.float32),
        ],
        scratch_shapes=[
            pltpu.VMEM((N, N), jnp.bfloat16),
            pltpu.VMEM((N, D_IN), jnp.float32),
            pltpu.VMEM((N, D_IN), jnp.bfloat16),
            pltpu.VMEM((KAN_K, D_OUT), jnp.bfloat16),
            pltpu.VMEM((N, CODE), jnp.float32),
            pltpu.VMEM((N, CODE), jnp.bfloat16),
        ],
        compiler_params=pltpu.CompilerParams(
            dimension_semantics=("arbitrary",)),
    )(G, G, x, W1, row(b1), W2, row(b2), row(ln_w), row(ln_b), W3p, row(b3))
    return (code, feat_out)
